# baseline (device time: 85063 ns/iter reference)
import jax
import jax.numpy as jnp
from jax import lax
from jax.experimental import pallas as pl
from jax.experimental.pallas import tpu as pltpu

N_DEV = 4


def kernel(partial, resid, gamma):
    m, d = partial.shape[-2], partial.shape[-1]
    partial = partial.reshape(m, d)
    gamma2 = gamma.reshape(1, d)

    def body(x_ref, r_ref, g_ref, o_ref, comm_ref, send_sems, recv_sems):
        my = lax.axis_index("i")
        left = lax.rem(my + N_DEV - 1, N_DEV)
        right = lax.rem(my + 1, N_DEV)

        barrier_sem = pltpu.get_barrier_semaphore()
        for nbr in (left, right):
            pl.semaphore_signal(
                barrier_sem, inc=1,
                device_id=(nbr,), device_id_type=pl.DeviceIdType.MESH,
            )
        pl.semaphore_wait(barrier_sem, 2)

        comm_ref[0] = x_ref[...].astype(jnp.bfloat16)
        acc = x_ref[...]
        for h in range(N_DEV - 1):
            rdma = pltpu.make_async_remote_copy(
                src_ref=comm_ref.at[h],
                dst_ref=comm_ref.at[h + 1],
                send_sem=send_sems.at[h],
                recv_sem=recv_sems.at[h + 1],
                device_id=(right,),
                device_id_type=pl.DeviceIdType.MESH,
            )
            rdma.start()
            rdma.wait()
            acc = acc + comm_ref[h + 1].astype(jnp.float32)

        y = acc + r_ref[...]
        rms = jnp.sqrt(jnp.mean(y * y, axis=-1, keepdims=True) + 1e-6)
        o_ref[...] = y / rms * g_ref[...]

    return pl.pallas_call(
        body,
        out_shape=jax.ShapeDtypeStruct((m, d), jnp.float32),
        in_specs=[pl.BlockSpec(memory_space=pltpu.VMEM)] * 3,
        out_specs=pl.BlockSpec(memory_space=pltpu.VMEM),
        scratch_shapes=[
            pltpu.VMEM((N_DEV, m, d), jnp.bfloat16),
            pltpu.SemaphoreType.DMA((N_DEV,)),
            pltpu.SemaphoreType.DMA((N_DEV,)),
        ],
        compiler_params=pltpu.CompilerParams(collective_id=0),
    )(partial, resid, gamma2)


# device time: 34842 ns/iter; 2.4414x vs baseline; 2.4414x over previous
import jax
import jax.numpy as jnp
from jax import lax
from jax.experimental import pallas as pl
from jax.experimental.pallas import tpu as pltpu

N_DEV = 4


def kernel(partial, resid, gamma):
    m, d = partial.shape[-2], partial.shape[-1]
    partial = partial.reshape(m, d)
    gamma2 = gamma.reshape(1, d)
    mh, mq, dh = m // 2, m // 4, d // 2

    def body(x_ref, r_ref, g_ref, o_ref, work_ref,
             recv_a1, recv_b1, recv_a2, recv_b2, send_sems, recv_sems):
        my = lax.axis_index("i")
        p1 = my ^ 1
        p2 = 3 - my

        barrier_sem = pltpu.get_barrier_semaphore()
        for nbr in (p1, p2):
            pl.semaphore_signal(
                barrier_sem, inc=1,
                device_id=(nbr,), device_id_type=pl.DeviceIdType.MESH,
            )
        pl.semaphore_wait(barrier_sem, 2)

        keep_a_low = (my == 0) | (my == 3)
        keep_b_low = my < 2
        h_a = jnp.where(keep_a_low, 0, mh)
        h_b = jnp.where(keep_b_low, 0, mh)
        q_a = h_a + jnp.where(my < p2, 0, mq)
        q_b = h_b + jnp.where(my < p1, 0, mq)

        cols_a = pl.ds(0, dh)
        cols_b = pl.ds(dh, dh)

        work_ref[...] = x_ref[...].astype(jnp.bfloat16)

        def exchange(srcs_dsts_sems_devs):
            rdmas = []
            for src, dst, s, r, dev in srcs_dsts_sems_devs:
                rdma = pltpu.make_async_remote_copy(
                    src_ref=src, dst_ref=dst,
                    send_sem=send_sems.at[s], recv_sem=recv_sems.at[r],
                    device_id=(dev,), device_id_type=pl.DeviceIdType.MESH,
                )
                rdma.start()
                rdmas.append(rdma)
            for rdma in rdmas:
                rdma.wait()

        exchange([
            (work_ref.at[pl.ds(mh - h_a, mh), cols_a], recv_a1.at[...], 0, 0, p1),
            (work_ref.at[pl.ds(mh - h_b, mh), cols_b], recv_b1.at[...], 1, 1, p2),
        ])
        work_ref[pl.ds(h_a, mh), cols_a] = (
            work_ref[pl.ds(h_a, mh), cols_a] + recv_a1[...]
        )
        work_ref[pl.ds(h_b, mh), cols_b] = (
            work_ref[pl.ds(h_b, mh), cols_b] + recv_b1[...]
        )

        send_q_a = h_a + mq - (q_a - h_a)
        send_q_b = h_b + mq - (q_b - h_b)
        exchange([
            (work_ref.at[pl.ds(send_q_a, mq), cols_a], recv_a2.at[...], 2, 2, p2),
            (work_ref.at[pl.ds(send_q_b, mq), cols_b], recv_b2.at[...], 3, 3, p1),
        ])
        work_ref[pl.ds(q_a, mq), cols_a] = (
            work_ref[pl.ds(q_a, mq), cols_a] + recv_a2[...]
            + r_ref[pl.ds(q_a, mq), cols_a].astype(jnp.bfloat16)
        )
        work_ref[pl.ds(q_b, mq), cols_b] = (
            work_ref[pl.ds(q_b, mq), cols_b] + recv_b2[...]
            + r_ref[pl.ds(q_b, mq), cols_b].astype(jnp.bfloat16)
        )

        exchange([
            (work_ref.at[pl.ds(q_a, mq), cols_a],
             work_ref.at[pl.ds(q_a, mq), cols_a], 4, 4, p2),
            (work_ref.at[pl.ds(q_b, mq), cols_b],
             work_ref.at[pl.ds(q_b, mq), cols_b], 5, 5, p1),
        ])

        exchange([
            (work_ref.at[pl.ds(h_a, mh), cols_a],
             work_ref.at[pl.ds(h_a, mh), cols_a], 6, 6, p1),
            (work_ref.at[pl.ds(h_b, mh), cols_b],
             work_ref.at[pl.ds(h_b, mh), cols_b], 7, 7, p2),
        ])

        y = work_ref[...].astype(jnp.float32)
        rms = jnp.sqrt(jnp.mean(y * y, axis=-1, keepdims=True) + 1e-6)
        o_ref[...] = y / rms * g_ref[...]

    return pl.pallas_call(
        body,
        out_shape=jax.ShapeDtypeStruct((m, d), jnp.float32),
        in_specs=[pl.BlockSpec(memory_space=pltpu.VMEM)] * 3,
        out_specs=pl.BlockSpec(memory_space=pltpu.VMEM),
        scratch_shapes=[
            pltpu.VMEM((m, d), jnp.bfloat16),
            pltpu.VMEM((mh, dh), jnp.bfloat16),
            pltpu.VMEM((mh, dh), jnp.bfloat16),
            pltpu.VMEM((mq, dh), jnp.bfloat16),
            pltpu.VMEM((mq, dh), jnp.bfloat16),
            pltpu.SemaphoreType.DMA((8,)),
            pltpu.SemaphoreType.DMA((8,)),
        ],
        compiler_params=pltpu.CompilerParams(collective_id=0),
    )(partial, resid, gamma2)


# device time: 28731 ns/iter; 2.9607x vs baseline; 1.2127x over previous
import jax
import jax.numpy as jnp
from jax import lax
from jax.experimental import pallas as pl
from jax.experimental.pallas import tpu as pltpu

N_DEV = 4
SUB = 4


def kernel(partial, resid, gamma):
    m, d = partial.shape[-2], partial.shape[-1]
    partial = partial.reshape(m, d)
    gamma2 = gamma.reshape(1, d)
    blk = m // 4
    sb = blk // SUB

    def body(x_hbm, r_hbm, g_ref, o_hbm, x_vm, rv_vm, o_vm, work_ref,
             recv_a1, recv_b1, recv_a2, recv_b2,
             send_sems, recv_sems, dma_sems):
        my = lax.axis_index("i")
        p1 = my ^ 1
        p2 = 3 - my

        a_off = jnp.where((my == 0) | (my == 3), 0, blk)
        a_send = blk - a_off
        b_low = jnp.where(my < 2, 0, blk)
        b_off = 2 * blk + b_low
        b_send = 3 * blk - b_low

        def rows(off):
            return pl.ds(off, blk)

        def srows(off, c):
            return pl.ds(off + c * sb, sb)

        def fetch(src, dst, sem):
            cp = pltpu.make_async_copy(src, dst, dma_sems.at[sem])
            cp.start()
            return cp

        f_xa = fetch(x_hbm.at[rows(a_send), :], x_vm.at[rows(a_send), :], 0)
        f_xb = fetch(x_hbm.at[rows(b_send), :], x_vm.at[rows(b_send), :], 1)
        f_xa2 = fetch(x_hbm.at[rows(a_off), :], x_vm.at[rows(a_off), :], 2)
        f_xb2 = fetch(x_hbm.at[rows(b_off), :], x_vm.at[rows(b_off), :], 3)
        f_ra = fetch(r_hbm.at[rows(a_off), :], rv_vm.at[pl.ds(0, blk), :], 4)
        f_rb = fetch(r_hbm.at[rows(b_off), :], rv_vm.at[pl.ds(blk, blk), :], 5)

        barrier_sem = pltpu.get_barrier_semaphore()
        for nbr in (p1, p2):
            pl.semaphore_signal(
                barrier_sem, inc=1,
                device_id=(nbr,), device_id_type=pl.DeviceIdType.MESH,
            )
        pl.semaphore_wait(barrier_sem, 2)

        def start(src, dst, sem, dev):
            rdma = pltpu.make_async_remote_copy(
                src_ref=src, dst_ref=dst,
                send_sem=send_sems.at[sem], recv_sem=recv_sems.at[sem],
                device_id=(dev,), device_id_type=pl.DeviceIdType.MESH,
            )
            rdma.start()
            return rdma

        def sem_idx(p, s, c):
            return (p * 2 + s) * SUB + c

        f_xa.wait()
        f_xb.wait()
        ph1a, ph1b = [], []
        for c in range(SUB):
            work_ref[srows(a_send, c), :] = (
                x_vm[srows(a_send, c), :].astype(jnp.bfloat16)
            )
            ph1a.append(start(work_ref.at[srows(a_send, c), :],
                              recv_a1.at[srows(0, c), :], sem_idx(0, 0, c), p1))
            work_ref[srows(b_send, c), :] = (
                x_vm[srows(b_send, c), :].astype(jnp.bfloat16)
            )
            ph1b.append(start(work_ref.at[srows(b_send, c), :],
                              recv_b1.at[srows(0, c), :], sem_idx(0, 1, c), p2))
        f_xa2.wait()
        work_ref[rows(a_off), :] = x_vm[rows(a_off), :].astype(jnp.bfloat16)
        f_xb2.wait()
        work_ref[rows(b_off), :] = x_vm[rows(b_off), :].astype(jnp.bfloat16)

        ph2a, ph2b = [], []
        for c in range(SUB):
            ph1a[c].wait()
            work_ref[srows(a_off, c), :] = (
                work_ref[srows(a_off, c), :] + recv_a1[srows(0, c), :]
            )
            ph2a.append(start(work_ref.at[srows(a_off, c), :],
                              recv_a2.at[srows(0, c), :], sem_idx(1, 0, c), p2))
            ph1b[c].wait()
            work_ref[srows(b_off, c), :] = (
                work_ref[srows(b_off, c), :] + recv_b1[srows(0, c), :]
            )
            ph2b.append(start(work_ref.at[srows(b_off, c), :],
                              recv_b2.at[srows(0, c), :], sem_idx(1, 1, c), p1))
        f_ra.wait()
        f_rb.wait()

        def norm_rows(off, recv, r_base, c):
            y = (work_ref[srows(off, c), :]
                 + recv[srows(0, c), :]).astype(jnp.float32)
            y = y + rv_vm[srows(r_base, c), :]
            rms = jnp.sqrt(jnp.mean(y * y, axis=-1, keepdims=True) + 1e-6)
            out = y / rms * g_ref[...]
            work_ref[srows(off, c), :] = out.astype(jnp.bfloat16)
            return out

        ph3a, ph3b = [], []
        w_out = []
        for c in range(SUB):
            ph2a[c].wait()
            o_vm[srows(a_off, c), :] = norm_rows(a_off, recv_a2, 0, c)
            ph3a.append(start(work_ref.at[srows(a_off, c), :],
                              work_ref.at[srows(a_off, c), :],
                              sem_idx(2, 0, c), p1))
            w_out.append(fetch(o_vm.at[srows(a_off, c), :],
                               o_hbm.at[srows(a_off, c), :], 6 + c))
            ph2b[c].wait()
            o_vm[srows(b_off, c), :] = norm_rows(b_off, recv_b2, blk, c)
            ph3b.append(start(work_ref.at[srows(b_off, c), :],
                              work_ref.at[srows(b_off, c), :],
                              sem_idx(2, 1, c), p2))
            w_out.append(fetch(o_vm.at[srows(b_off, c), :],
                               o_hbm.at[srows(b_off, c), :], 6 + SUB + c))

        for c in range(SUB):
            ph3a[c].wait()
            o_vm[srows(a_send, c), :] = (
                work_ref[srows(a_send, c), :].astype(jnp.float32)
            )
            w_out.append(fetch(o_vm.at[srows(a_send, c), :],
                               o_hbm.at[srows(a_send, c), :], 6 + 2 * SUB + c))
            ph3b[c].wait()
            o_vm[srows(b_send, c), :] = (
                work_ref[srows(b_send, c), :].astype(jnp.float32)
            )
            w_out.append(fetch(o_vm.at[srows(b_send, c), :],
                               o_hbm.at[srows(b_send, c), :], 6 + 3 * SUB + c))
        for w in w_out:
            w.wait()

    return pl.pallas_call(
        body,
        out_shape=jax.ShapeDtypeStruct((m, d), jnp.float32),
        in_specs=[
            pl.BlockSpec(memory_space=pl.ANY),
            pl.BlockSpec(memory_space=pl.ANY),
            pl.BlockSpec(memory_space=pltpu.VMEM),
        ],
        out_specs=pl.BlockSpec(memory_space=pl.ANY),
        scratch_shapes=[
            pltpu.VMEM((m, d), jnp.float32),
            pltpu.VMEM((m // 2, d), jnp.float32),
            pltpu.VMEM((m, d), jnp.float32),
            pltpu.VMEM((m, d), jnp.bfloat16),
            pltpu.VMEM((blk, d), jnp.bfloat16),
            pltpu.VMEM((blk, d), jnp.bfloat16),
            pltpu.VMEM((blk, d), jnp.bfloat16),
            pltpu.VMEM((blk, d), jnp.bfloat16),
            pltpu.SemaphoreType.DMA((6 * SUB,)),
            pltpu.SemaphoreType.DMA((6 * SUB,)),
            pltpu.SemaphoreType.DMA((6 + 4 * SUB,)),
        ],
        compiler_params=pltpu.CompilerParams(collective_id=0),
    )(partial, resid, gamma2)
